# baseline (speedup 1.0000x reference)
import functools

import numpy as np

import jax
import jax.numpy as jnp
from jax import lax
from jax.experimental import pallas as pl
from jax.experimental.pallas import tpu as pltpu, tpu_sc as plsc

N_NODES = 10000
N_EDGES = 320000
D_FEAT = 128
D_EDGE = 16
HIDDEN = 64
N_CLASSES = 2

def _lane_take(v, idx):
    dn = lax.GatherDimensionNumbers(
        offset_dims=(), collapsed_slice_dims=(0,), start_index_map=(0,))
    return lax.gather(v, idx[:, None], dn, (1,),
                      mode=lax.GatherScatterMode.PROMISE_IN_BOUNDS)


def _allsum(v, iota):
    for sh in (8, 4, 2, 1):
        v = v + _lane_take(v, jnp.bitwise_xor(iota, sh))
    return v


NW = 32
EDGES_PER_TILE = N_EDGES // NW
CHUNK = 80
N_CHUNKS = EDGES_PER_TILE // CHUNK
L = 16
DUPW = 1024


def _node_proj_body(x_ref, w1s_ref, w1d_ref, a_ref, b_ref):
    xv = x_ref[...]
    a_ref[...] = jnp.dot(xv, w1s_ref[...], preferred_element_type=jnp.float32)
    b_ref[...] = jnp.dot(xv, w1d_ref[...], preferred_element_type=jnp.float32)


def _node_proj(x, w1s, w1d):
    blk = 2000
    grid = (N_NODES // blk,)
    return pl.pallas_call(
        _node_proj_body,
        grid=grid,
        in_specs=[
            pl.BlockSpec((blk, D_FEAT), lambda i: (i, 0)),
            pl.BlockSpec((D_FEAT, HIDDEN), lambda i: (0, 0)),
            pl.BlockSpec((D_FEAT, HIDDEN), lambda i: (0, 0)),
        ],
        out_specs=[
            pl.BlockSpec((blk, HIDDEN), lambda i: (i, 0)),
            pl.BlockSpec((blk, HIDDEN), lambda i: (i, 0)),
        ],
        out_shape=[
            jax.ShapeDtypeStruct((N_NODES, HIDDEN), jnp.float32),
            jax.ShapeDtypeStruct((N_NODES, HIDDEN), jnp.float32),
        ],
    )(x, w1s, w1d)


def _edge_proj_body(ea_ref, w1e_ref, b1_ref, c_ref):
    c_ref[...] = (
        jnp.dot(ea_ref[...], w1e_ref[...], preferred_element_type=jnp.float32)
        + b1_ref[...]
    )


def _edge_proj(ea, w1e, b1row):
    blk = 20000
    grid = (N_EDGES // blk,)
    return pl.pallas_call(
        _edge_proj_body,
        grid=grid,
        in_specs=[
            pl.BlockSpec((blk, D_EDGE), lambda i: (i, 0)),
            pl.BlockSpec((D_EDGE, HIDDEN), lambda i: (0, 0)),
            pl.BlockSpec((1, HIDDEN), lambda i: (0, 0)),
        ],
        out_specs=pl.BlockSpec((blk, HIDDEN), lambda i: (i, 0)),
        out_shape=jax.ShapeDtypeStruct((N_EDGES, HIDDEN), jnp.float32),
    )(ea, w1e, b1row)


def _merge_body(p_ref, o_ref):
    o_ref[...] = jnp.max(p_ref[...], axis=0)


def _merge_partials(partials3):
    return pl.pallas_call(
        _merge_body,
        out_shape=jax.ShapeDtypeStruct((8, N_NODES // 8), jnp.float32),
    )(partials3)


def _sc_edge_kernel(a_hbm, b_hbm, c_hbm, src_hbm, dst_hbm, w2t_hbm, b2_hbm,
                    pred0_hbm, pred1_hbm, part_hbm,
                    srcall_v, dstall_v, a3, b3, c3, p03, p13,
                    w2_v, b2_v, best_v, acc0_v, acc1_v,
                    sem_in0, sem_in1, sem_out0, sem_out1):
    wid = lax.axis_index("s") * 2 + lax.axis_index("c")
    tile_base = wid * EDGES_PER_TILE
    sem_in = (sem_in0, sem_in1)
    sem_out = (sem_out0, sem_out1)

    pltpu.sync_copy(w2t_hbm, w2_v)
    pltpu.sync_copy(b2_hbm, b2_v)
    pltpu.sync_copy(src_hbm.at[pl.ds(tile_base, EDGES_PER_TILE)], srcall_v)
    pltpu.sync_copy(dst_hbm.at[pl.ds(tile_base, EDGES_PER_TILE)], dstall_v)
    iota = lax.iota(jnp.int32, L)
    b2vec = b2_v[...]
    b2_0v = _lane_take(b2vec, jnp.zeros((L,), jnp.int32))
    b2_1v = _lane_take(b2vec, jnp.ones((L,), jnp.int32))
    w2c0 = [w2_v[0, pl.ds(q * L, L)] for q in range(HIDDEN // L)]
    w2c1 = [w2_v[1, pl.ds(q * L, L)] for q in range(HIDDEN // L)]

    ninf = jnp.full((L,), -jnp.inf, dtype=jnp.float32)

    def init_body(i, _):
        best_v[pl.ds(i * L, L)] = ninf
        return 0

    lax.fori_loop(0, N_NODES // L, init_body, 0)

    rot_idx = [jnp.bitwise_and(iota + r, L - 1) for r in range(1, L)]
    iotaL = iota * L

    def issue(g, bp):
        esl = pl.ds(g * CHUNK, CHUNK)
        pltpu.async_copy(a_hbm.at[srcall_v.at[esl]], a3.at[bp], sem_in[bp])
        pltpu.async_copy(b_hbm.at[dstall_v.at[esl]], b3.at[bp], sem_in[bp])
        pltpu.async_copy(c_hbm.at[pl.ds(tile_base + g * CHUNK, CHUNK)],
                         c3.at[bp], sem_in[bp])

    def drain_in(bp):
        dummy_f32 = a_hbm.at[pl.ds(0, CHUNK)]
        pltpu.make_async_copy(dummy_f32, a3.at[bp], sem_in[bp]).wait()
        pltpu.make_async_copy(dummy_f32, b3.at[bp], sem_in[bp]).wait()
        pltpu.make_async_copy(c_hbm.at[pl.ds(0, CHUNK)], c3.at[bp],
                              sem_in[bp]).wait()

    def drain_out(bp):
        dummy = pred0_hbm.at[pl.ds(0, CHUNK)]
        pltpu.make_async_copy(dummy, p03.at[bp], sem_out[bp]).wait()
        pltpu.make_async_copy(dummy, p13.at[bp], sem_out[bp]).wait()

    def compute(g, bp):
        @pl.when(g >= 2)
        def _():
            drain_out(bp)

        for j in range(CHUNK // L):
            def edge_body(i, _):
                e = j * L + i
                acc0 = jnp.zeros((L,), jnp.float32)
                acc1 = jnp.zeros((L,), jnp.float32)
                for q in range(HIDDEN // L):
                    sl = pl.ds(q * L, L)
                    h = jnp.maximum(
                        a3[bp, e, sl] + b3[bp, e, sl] + c3[bp, e, sl],
                        0.0)
                    acc0 = acc0 + h * w2c0[q]
                    acc1 = acc1 + h * w2c1[q]
                idxv = iotaL + i
                plsc.store_scatter(acc0_v, [idxv], acc0)
                plsc.store_scatter(acc1_v, [idxv], acc1)
                return 0

            lax.fori_loop(0, L, edge_body, 0)
            p0g = b2_0v
            p1g = b2_1v
            for r in range(L):
                rsl = pl.ds(r * L, L)
                p0g = p0g + acc0_v[rsl]
                p1g = p1g + acc1_v[rsl]
            sl = pl.ds(j * L, L)
            p03[bp, sl] = p0g
            p13[bp, sl] = p1g

            score = 1.0 / (1.0 + jnp.exp(-p1g))
            k = dstall_v[pl.ds(g * CHUNK + j * L, L)]
            v = score
            dup_earlier = jnp.zeros((L,), jnp.bool_)
            for r in range(1, L):
                kr = _lane_take(k, rot_idx[r - 1])
                vr = _lane_take(v, rot_idx[r - 1])
                same = kr == k
                v = jnp.where(same, jnp.maximum(v, vr), v)
                dup_earlier = dup_earlier | (same & (iota + r >= L))
            old = plsc.load_gather(best_v, [k])
            plsc.store_scatter(best_v, [k], jnp.maximum(old, v),
                               mask=jnp.logical_not(dup_earlier))

        base = tile_base + g * CHUNK
        pltpu.async_copy(p03.at[bp], pred0_hbm.at[pl.ds(base, CHUNK)],
                         sem_out[bp])
        pltpu.async_copy(p13.at[bp], pred1_hbm.at[pl.ds(base, CHUNK)],
                         sem_out[bp])

    issue(0, 0)

    def pair_body(kk, _):
        g = 2 * kk
        issue(g + 1, 1)
        drain_in(0)
        compute(g, 0)
        issue(g + 2, 0)
        drain_in(1)
        compute(g + 1, 1)
        return 0

    lax.fori_loop(0, (N_CHUNKS - 1) // 2, pair_body, 0)
    drain_in(0)
    compute(N_CHUNKS - 1, 0)
    drain_out(0)
    drain_out(1)
    pltpu.sync_copy(best_v, part_hbm.at[wid])


def _sc_edge(A, B, C, src, dst, w2t, b2):
    mesh = plsc.VectorSubcoreMesh(core_axis_name="c", subcore_axis_name="s")
    f32 = jnp.float32
    kern = functools.partial(
        pl.kernel,
        mesh=mesh,
        compiler_params=pltpu.CompilerParams(
            needs_layout_passes=False, use_tc_tiling_on_sc=False),
        out_type=[
            jax.ShapeDtypeStruct((N_EDGES,), f32),
            jax.ShapeDtypeStruct((N_EDGES,), f32),
            jax.ShapeDtypeStruct((NW, N_NODES), f32),
        ],
        scratch_types=[
            pltpu.VMEM((EDGES_PER_TILE,), jnp.int32),
            pltpu.VMEM((EDGES_PER_TILE,), jnp.int32),
            pltpu.VMEM((2, CHUNK, HIDDEN), f32),
            pltpu.VMEM((2, CHUNK, HIDDEN), f32),
            pltpu.VMEM((2, CHUNK, HIDDEN), f32),
            pltpu.VMEM((2, CHUNK), f32),
            pltpu.VMEM((2, CHUNK), f32),
            pltpu.VMEM((N_CLASSES, HIDDEN), f32),
            pltpu.VMEM((L,), f32),
            pltpu.VMEM((N_NODES,), f32),
            pltpu.VMEM((L * L,), f32),
            pltpu.VMEM((L * L,), f32),
            pltpu.SemaphoreType.DMA,
            pltpu.SemaphoreType.DMA,
            pltpu.SemaphoreType.DMA,
            pltpu.SemaphoreType.DMA,
        ],
    )(_sc_edge_kernel)
    return kern(A, B, C, src, dst, w2t, b2)


def kernel(x, edge_index, edge_attr, W1, b1, W2, b2):
    src = edge_index[0].astype(jnp.int32)
    dst = edge_index[1].astype(jnp.int32)
    w1s = W1[:D_FEAT]
    w1d = W1[D_FEAT:2 * D_FEAT]
    w1e = W1[2 * D_FEAT:]

    A, B = _node_proj(x, w1s, w1d)
    C = _edge_proj(edge_attr, w1e, b1.reshape(1, HIDDEN))
    b2pad = jnp.pad(b2.astype(jnp.float32), (0, L - N_CLASSES))
    p0, p1, partials = _sc_edge(A, B, C, src, dst,
                                W2.T.astype(jnp.float32), b2pad)
    node_best = _merge_partials(
        partials.reshape(NW, 8, N_NODES // 8)).reshape(N_NODES)
    edge_pred = jnp.stack([p0, p1], axis=1)
    return edge_pred, node_best


# speedup vs baseline: 1.1217x; 1.1217x over previous
import functools

import numpy as np

import jax
import jax.numpy as jnp
from jax import lax
from jax.experimental import pallas as pl
from jax.experimental.pallas import tpu as pltpu, tpu_sc as plsc

N_NODES = 10000
N_EDGES = 320000
D_FEAT = 128
D_EDGE = 16
HIDDEN = 64
N_CLASSES = 2

def _lane_take(v, idx):
    dn = lax.GatherDimensionNumbers(
        offset_dims=(), collapsed_slice_dims=(0,), start_index_map=(0,))
    return lax.gather(v, idx[:, None], dn, (1,),
                      mode=lax.GatherScatterMode.PROMISE_IN_BOUNDS)


def _allsum(v, iota):
    for sh in (8, 4, 2, 1):
        v = v + _lane_take(v, jnp.bitwise_xor(iota, sh))
    return v


NW = 32
EDGES_PER_TILE = N_EDGES // NW
CHUNK = 80
N_CHUNKS = EDGES_PER_TILE // CHUNK
L = 16
DUPW = 1024


def _node_proj_body(x_ref, w1s_ref, w1d_ref, a_ref, b_ref):
    xv = x_ref[...]
    a_ref[...] = jnp.dot(xv, w1s_ref[...], preferred_element_type=jnp.float32)
    b_ref[...] = jnp.dot(xv, w1d_ref[...], preferred_element_type=jnp.float32)


def _node_proj(x, w1s, w1d):
    blk = 2000
    grid = (N_NODES // blk,)
    return pl.pallas_call(
        _node_proj_body,
        grid=grid,
        in_specs=[
            pl.BlockSpec((blk, D_FEAT), lambda i: (i, 0)),
            pl.BlockSpec((D_FEAT, HIDDEN), lambda i: (0, 0)),
            pl.BlockSpec((D_FEAT, HIDDEN), lambda i: (0, 0)),
        ],
        out_specs=[
            pl.BlockSpec((blk, HIDDEN), lambda i: (i, 0)),
            pl.BlockSpec((blk, HIDDEN), lambda i: (i, 0)),
        ],
        out_shape=[
            jax.ShapeDtypeStruct((N_NODES, HIDDEN), jnp.float32),
            jax.ShapeDtypeStruct((N_NODES, HIDDEN), jnp.float32),
        ],
    )(x, w1s, w1d)


def _edge_proj_body(ea_ref, w1e_ref, b1_ref, c_ref):
    c_ref[...] = (
        jnp.dot(ea_ref[...], w1e_ref[...], preferred_element_type=jnp.float32)
        + b1_ref[...]
    )


def _edge_proj(ea, w1e, b1row):
    blk = 20000
    grid = (N_EDGES // blk,)
    return pl.pallas_call(
        _edge_proj_body,
        grid=grid,
        in_specs=[
            pl.BlockSpec((blk, D_EDGE), lambda i: (i, 0)),
            pl.BlockSpec((D_EDGE, HIDDEN), lambda i: (0, 0)),
            pl.BlockSpec((1, HIDDEN), lambda i: (0, 0)),
        ],
        out_specs=pl.BlockSpec((blk, HIDDEN), lambda i: (i, 0)),
        out_shape=jax.ShapeDtypeStruct((N_EDGES, HIDDEN), jnp.float32),
    )(ea, w1e, b1row)


def _merge_body(p_ref, o_ref):
    o_ref[...] = jnp.max(p_ref[...], axis=0)


def _merge_partials(partials3):
    return pl.pallas_call(
        _merge_body,
        out_shape=jax.ShapeDtypeStruct((8, N_NODES // 8), jnp.float32),
    )(partials3)


def _sc_edge_kernel(a_hbm, b_hbm, c_hbm, src_hbm, dst_hbm, w2t_hbm, b2_hbm,
                    pred0_hbm, pred1_hbm, part_hbm,
                    srcall_v, dstall_v, a3, b3, c3, p03, p13,
                    w2_v, b2_v, best_v,
                    sem_in0, sem_in1, sem_out0, sem_out1):
    wid = lax.axis_index("s") * 2 + lax.axis_index("c")
    tile_base = wid * EDGES_PER_TILE
    sem_in = (sem_in0, sem_in1)
    sem_out = (sem_out0, sem_out1)

    pltpu.sync_copy(w2t_hbm, w2_v)
    pltpu.sync_copy(b2_hbm, b2_v)
    pltpu.sync_copy(src_hbm.at[pl.ds(tile_base, EDGES_PER_TILE)], srcall_v)
    pltpu.sync_copy(dst_hbm.at[pl.ds(tile_base, EDGES_PER_TILE)], dstall_v)
    iota = lax.iota(jnp.int32, L)
    b2vec = b2_v[...]
    b2_0v = _lane_take(b2vec, jnp.zeros((L,), jnp.int32))
    b2_1v = _lane_take(b2vec, jnp.ones((L,), jnp.int32))
    w2c0 = [w2_v[0, pl.ds(q * L, L)] for q in range(HIDDEN // L)]
    w2c1 = [w2_v[1, pl.ds(q * L, L)] for q in range(HIDDEN // L)]

    ninf = jnp.full((L,), -jnp.inf, dtype=jnp.float32)

    def init_body(i, _):
        best_v[pl.ds(i * L, L)] = ninf
        return 0

    lax.fori_loop(0, N_NODES // L, init_body, 0)

    rot_idx = [jnp.bitwise_and(iota + r, L - 1) for r in range(1, L)]
    iotaL = iota * L

    def issue(g, bp):
        esl = pl.ds(g * CHUNK, CHUNK)
        pltpu.async_copy(a_hbm.at[srcall_v.at[esl]], a3.at[bp], sem_in[bp])
        pltpu.async_copy(b_hbm.at[dstall_v.at[esl]], b3.at[bp], sem_in[bp])
        pltpu.async_copy(c_hbm.at[pl.ds(tile_base + g * CHUNK, CHUNK)],
                         c3.at[bp], sem_in[bp])

    def drain_in(bp):
        dummy_f32 = a_hbm.at[pl.ds(0, CHUNK)]
        pltpu.make_async_copy(dummy_f32, a3.at[bp], sem_in[bp]).wait()
        pltpu.make_async_copy(dummy_f32, b3.at[bp], sem_in[bp]).wait()
        pltpu.make_async_copy(c_hbm.at[pl.ds(0, CHUNK)], c3.at[bp],
                              sem_in[bp]).wait()

    def drain_out(bp):
        dummy = pred0_hbm.at[pl.ds(0, CHUNK)]
        pltpu.make_async_copy(dummy, p03.at[bp], sem_out[bp]).wait()
        pltpu.make_async_copy(dummy, p13.at[bp], sem_out[bp]).wait()

    def compute(g, bp):
        @pl.when(g >= 2)
        def _():
            drain_out(bp)

        for j in range(CHUNK // L):
            def edge_body(i, carry):
                p0vec, p1vec = carry
                e = j * L + i
                acc0 = jnp.zeros((L,), jnp.float32)
                acc1 = jnp.zeros((L,), jnp.float32)
                for q in range(HIDDEN // L):
                    sl = pl.ds(q * L, L)
                    h = jnp.maximum(
                        a3[bp, e, sl] + b3[bp, e, sl] + c3[bp, e, sl],
                        0.0)
                    acc0 = acc0 + h * w2c0[q]
                    acc1 = acc1 + h * w2c1[q]
                m = iota == i
                p0vec = jnp.where(m, _allsum(acc0, iota), p0vec)
                p1vec = jnp.where(m, _allsum(acc1, iota), p1vec)
                return p0vec, p1vec

            zero = jnp.zeros((L,), jnp.float32)
            p0g, p1g = lax.fori_loop(0, L, edge_body, (zero, zero))
            p0g = p0g + b2_0v
            p1g = p1g + b2_1v
            sl = pl.ds(j * L, L)
            p03[bp, sl] = p0g
            p13[bp, sl] = p1g

            score = 1.0 / (1.0 + jnp.exp(-p1g))
            k = dstall_v[pl.ds(g * CHUNK + j * L, L)]
            v = score
            dup_earlier = jnp.zeros((L,), jnp.bool_)
            for r in range(1, L):
                kr = _lane_take(k, rot_idx[r - 1])
                vr = _lane_take(v, rot_idx[r - 1])
                same = kr == k
                v = jnp.where(same, jnp.maximum(v, vr), v)
                dup_earlier = dup_earlier | (same & (iota + r >= L))
            old = plsc.load_gather(best_v, [k])
            plsc.store_scatter(best_v, [k], jnp.maximum(old, v),
                               mask=jnp.logical_not(dup_earlier))

        base = tile_base + g * CHUNK
        pltpu.async_copy(p03.at[bp], pred0_hbm.at[pl.ds(base, CHUNK)],
                         sem_out[bp])
        pltpu.async_copy(p13.at[bp], pred1_hbm.at[pl.ds(base, CHUNK)],
                         sem_out[bp])

    issue(0, 0)

    def pair_body(kk, _):
        g = 2 * kk
        issue(g + 1, 1)
        drain_in(0)
        compute(g, 0)
        issue(g + 2, 0)
        drain_in(1)
        compute(g + 1, 1)
        return 0

    lax.fori_loop(0, (N_CHUNKS - 1) // 2, pair_body, 0)
    drain_in(0)
    compute(N_CHUNKS - 1, 0)
    drain_out(0)
    drain_out(1)
    pltpu.sync_copy(best_v, part_hbm.at[wid])


def _sc_edge(A, B, C, src, dst, w2t, b2):
    mesh = plsc.VectorSubcoreMesh(core_axis_name="c", subcore_axis_name="s")
    f32 = jnp.float32
    kern = functools.partial(
        pl.kernel,
        mesh=mesh,
        compiler_params=pltpu.CompilerParams(
            needs_layout_passes=False, use_tc_tiling_on_sc=False),
        out_type=[
            jax.ShapeDtypeStruct((N_EDGES,), f32),
            jax.ShapeDtypeStruct((N_EDGES,), f32),
            jax.ShapeDtypeStruct((NW, N_NODES), f32),
        ],
        scratch_types=[
            pltpu.VMEM((EDGES_PER_TILE,), jnp.int32),
            pltpu.VMEM((EDGES_PER_TILE,), jnp.int32),
            pltpu.VMEM((2, CHUNK, HIDDEN), f32),
            pltpu.VMEM((2, CHUNK, HIDDEN), f32),
            pltpu.VMEM((2, CHUNK, HIDDEN), f32),
            pltpu.VMEM((2, CHUNK), f32),
            pltpu.VMEM((2, CHUNK), f32),
            pltpu.VMEM((N_CLASSES, HIDDEN), f32),
            pltpu.VMEM((L,), f32),
            pltpu.VMEM((N_NODES,), f32),
            pltpu.SemaphoreType.DMA,
            pltpu.SemaphoreType.DMA,
            pltpu.SemaphoreType.DMA,
            pltpu.SemaphoreType.DMA,
        ],
    )(_sc_edge_kernel)
    return kern(A, B, C, src, dst, w2t, b2)


def kernel(x, edge_index, edge_attr, W1, b1, W2, b2):
    src = edge_index[0].astype(jnp.int32)
    dst = edge_index[1].astype(jnp.int32)
    w1s = W1[:D_FEAT]
    w1d = W1[D_FEAT:2 * D_FEAT]
    w1e = W1[2 * D_FEAT:]

    A, B = _node_proj(x, w1s, w1d)
    C = _edge_proj(edge_attr, w1e, b1.reshape(1, HIDDEN))
    b2pad = jnp.pad(b2.astype(jnp.float32), (0, L - N_CLASSES))
    p0, p1, partials = _sc_edge(A, B, C, src, dst,
                                W2.T.astype(jnp.float32), b2pad)
    node_best = _merge_partials(
        partials.reshape(NW, 8, N_NODES // 8)).reshape(N_NODES)
    edge_pred = jnp.stack([p0, p1], axis=1)
    return edge_pred, node_best


# speedup vs baseline: 1.3712x; 1.2224x over previous
import functools

import numpy as np

import jax
import jax.numpy as jnp
from jax import lax
from jax.experimental import pallas as pl
from jax.experimental.pallas import tpu as pltpu, tpu_sc as plsc

N_NODES = 10000
N_EDGES = 320000
D_FEAT = 128
D_EDGE = 16
HIDDEN = 64
N_CLASSES = 2

def _lane_take(v, idx):
    dn = lax.GatherDimensionNumbers(
        offset_dims=(), collapsed_slice_dims=(0,), start_index_map=(0,))
    return lax.gather(v, idx[:, None], dn, (1,),
                      mode=lax.GatherScatterMode.PROMISE_IN_BOUNDS)


def _allsum(v, iota):
    for sh in (8, 4, 2, 1):
        v = v + _lane_take(v, jnp.bitwise_xor(iota, sh))
    return v


NW = 32
EDGES_PER_TILE = N_EDGES // NW
CHUNK = 80
N_CHUNKS = EDGES_PER_TILE // CHUNK
L = 16
DUPW = 1024


def _node_proj_body(x_ref, w1s_ref, w1d_ref, a_ref, b_ref):
    xv = x_ref[...]
    a_ref[...] = jnp.dot(xv, w1s_ref[...], preferred_element_type=jnp.float32)
    b_ref[...] = jnp.dot(xv, w1d_ref[...], preferred_element_type=jnp.float32)


def _node_proj(x, w1s, w1d):
    blk = 2000
    grid = (N_NODES // blk,)
    return pl.pallas_call(
        _node_proj_body,
        grid=grid,
        in_specs=[
            pl.BlockSpec((blk, D_FEAT), lambda i: (i, 0)),
            pl.BlockSpec((D_FEAT, HIDDEN), lambda i: (0, 0)),
            pl.BlockSpec((D_FEAT, HIDDEN), lambda i: (0, 0)),
        ],
        out_specs=[
            pl.BlockSpec((blk, HIDDEN), lambda i: (i, 0)),
            pl.BlockSpec((blk, HIDDEN), lambda i: (i, 0)),
        ],
        out_shape=[
            jax.ShapeDtypeStruct((N_NODES, HIDDEN), jnp.float32),
            jax.ShapeDtypeStruct((N_NODES, HIDDEN), jnp.float32),
        ],
    )(x, w1s, w1d)


def _edge_proj_body(ea_ref, w1e_ref, b1_ref, c_ref):
    c_ref[...] = (
        jnp.dot(ea_ref[...], w1e_ref[...], preferred_element_type=jnp.float32)
        + b1_ref[...]
    )


def _edge_proj(ea2, w1e2, b1row2):
    blk = 10000
    rows = N_EDGES // 2
    grid = (rows // blk,)
    return pl.pallas_call(
        _edge_proj_body,
        grid=grid,
        in_specs=[
            pl.BlockSpec((blk, 2 * D_EDGE), lambda i: (i, 0)),
            pl.BlockSpec((2 * D_EDGE, 2 * HIDDEN), lambda i: (0, 0)),
            pl.BlockSpec((1, 2 * HIDDEN), lambda i: (0, 0)),
        ],
        out_specs=pl.BlockSpec((blk, 2 * HIDDEN), lambda i: (i, 0)),
        out_shape=jax.ShapeDtypeStruct((rows, 2 * HIDDEN), jnp.float32),
    )(ea2, w1e2, b1row2)


def _merge_body(p_ref, o_ref):
    o_ref[...] = jnp.max(p_ref[...], axis=0)


def _merge_partials(partials3):
    return pl.pallas_call(
        _merge_body,
        out_shape=jax.ShapeDtypeStruct((8, N_NODES // 8), jnp.float32),
    )(partials3)


def _sc_edge_kernel(a_hbm, b_hbm, c_hbm, src_hbm, dst_hbm, w2t_hbm, b2_hbm,
                    pred0_hbm, pred1_hbm, part_hbm,
                    srcall_v, dstall_v, a3, b3, c3, p03, p13,
                    w2_v, b2_v, best_v,
                    sem_in0, sem_in1, sem_out0, sem_out1):
    wid = lax.axis_index("s") * 2 + lax.axis_index("c")
    tile_base = wid * EDGES_PER_TILE
    sem_in = (sem_in0, sem_in1)
    sem_out = (sem_out0, sem_out1)

    pltpu.sync_copy(w2t_hbm, w2_v)
    pltpu.sync_copy(b2_hbm, b2_v)
    pltpu.sync_copy(src_hbm.at[pl.ds(tile_base, EDGES_PER_TILE)], srcall_v)
    pltpu.sync_copy(dst_hbm.at[pl.ds(tile_base, EDGES_PER_TILE)], dstall_v)
    iota = lax.iota(jnp.int32, L)
    b2vec = b2_v[...]
    b2_0v = _lane_take(b2vec, jnp.zeros((L,), jnp.int32))
    b2_1v = _lane_take(b2vec, jnp.ones((L,), jnp.int32))
    w2c0 = [w2_v[0, pl.ds(q * L, L)] for q in range(HIDDEN // L)]
    w2c1 = [w2_v[1, pl.ds(q * L, L)] for q in range(HIDDEN // L)]

    ninf = jnp.full((L,), -jnp.inf, dtype=jnp.float32)

    def init_body(i, _):
        best_v[pl.ds(i * L, L)] = ninf
        return 0

    lax.fori_loop(0, N_NODES // L, init_body, 0)

    rot_idx = [jnp.bitwise_and(iota + r, L - 1) for r in range(1, L)]
    iotaL = iota * L

    def issue(g, bp):
        esl = pl.ds(g * CHUNK, CHUNK)
        pltpu.async_copy(a_hbm.at[srcall_v.at[esl]], a3.at[bp], sem_in[bp])
        pltpu.async_copy(b_hbm.at[dstall_v.at[esl]], b3.at[bp], sem_in[bp])
        pltpu.async_copy(
            c_hbm.at[pl.ds((tile_base + g * CHUNK) // 2, CHUNK // 2)],
            c3.at[bp], sem_in[bp])

    def drain_in(bp):
        dummy_f32 = a_hbm.at[pl.ds(0, CHUNK)]
        pltpu.make_async_copy(dummy_f32, a3.at[bp], sem_in[bp]).wait()
        pltpu.make_async_copy(dummy_f32, b3.at[bp], sem_in[bp]).wait()
        pltpu.make_async_copy(c_hbm.at[pl.ds(0, CHUNK // 2)], c3.at[bp],
                              sem_in[bp]).wait()

    def drain_out(bp):
        dummy = pred0_hbm.at[pl.ds(0, CHUNK)]
        pltpu.make_async_copy(dummy, p03.at[bp], sem_out[bp]).wait()
        pltpu.make_async_copy(dummy, p13.at[bp], sem_out[bp]).wait()

    def compute(g, bp):
        @pl.when(g >= 2)
        def _():
            drain_out(bp)

        for j in range(CHUNK // L):
            def edge_body(i, carry):
                p0vec, p1vec = carry
                e = j * L + i
                erow = lax.shift_right_logical(e, 1)
                ecol = jnp.bitwise_and(e, 1) * HIDDEN
                acc0 = jnp.zeros((L,), jnp.float32)
                acc1 = jnp.zeros((L,), jnp.float32)
                for q in range(HIDDEN // L):
                    sl = pl.ds(q * L, L)
                    h = jnp.maximum(
                        a3[bp, e, sl] + b3[bp, e, sl]
                        + c3[bp, erow, pl.ds(ecol + q * L, L)],
                        0.0)
                    acc0 = acc0 + h * w2c0[q]
                    acc1 = acc1 + h * w2c1[q]
                m = iota == i
                p0vec = jnp.where(m, _allsum(acc0, iota), p0vec)
                p1vec = jnp.where(m, _allsum(acc1, iota), p1vec)
                return p0vec, p1vec

            zero = jnp.zeros((L,), jnp.float32)
            p0g, p1g = lax.fori_loop(0, L, edge_body, (zero, zero))
            p0g = p0g + b2_0v
            p1g = p1g + b2_1v
            sl = pl.ds(j * L, L)
            p03[bp, sl] = p0g
            p13[bp, sl] = p1g

            score = 1.0 / (1.0 + jnp.exp(-p1g))
            k = dstall_v[pl.ds(g * CHUNK + j * L, L)]
            v = score
            dup_earlier = jnp.zeros((L,), jnp.bool_)
            for r in range(1, L):
                kr = _lane_take(k, rot_idx[r - 1])
                vr = _lane_take(v, rot_idx[r - 1])
                same = kr == k
                v = jnp.where(same, jnp.maximum(v, vr), v)
                dup_earlier = dup_earlier | (same & (iota + r >= L))
            old = plsc.load_gather(best_v, [k])
            plsc.store_scatter(best_v, [k], jnp.maximum(old, v),
                               mask=jnp.logical_not(dup_earlier))

        base = tile_base + g * CHUNK
        pltpu.async_copy(p03.at[bp], pred0_hbm.at[pl.ds(base, CHUNK)],
                         sem_out[bp])
        pltpu.async_copy(p13.at[bp], pred1_hbm.at[pl.ds(base, CHUNK)],
                         sem_out[bp])

    issue(0, 0)

    def pair_body(kk, _):
        g = 2 * kk
        issue(g + 1, 1)
        drain_in(0)
        compute(g, 0)
        issue(g + 2, 0)
        drain_in(1)
        compute(g + 1, 1)
        return 0

    lax.fori_loop(0, (N_CHUNKS - 1) // 2, pair_body, 0)
    drain_in(0)
    compute(N_CHUNKS - 1, 0)
    drain_out(0)
    drain_out(1)
    pltpu.sync_copy(best_v, part_hbm.at[wid])


def _sc_edge(A, B, C, src, dst, w2t, b2):
    mesh = plsc.VectorSubcoreMesh(core_axis_name="c", subcore_axis_name="s")
    f32 = jnp.float32
    kern = functools.partial(
        pl.kernel,
        mesh=mesh,
        compiler_params=pltpu.CompilerParams(
            needs_layout_passes=False, use_tc_tiling_on_sc=False),
        out_type=[
            jax.ShapeDtypeStruct((N_EDGES,), f32),
            jax.ShapeDtypeStruct((N_EDGES,), f32),
            jax.ShapeDtypeStruct((NW, N_NODES), f32),
        ],
        scratch_types=[
            pltpu.VMEM((EDGES_PER_TILE,), jnp.int32),
            pltpu.VMEM((EDGES_PER_TILE,), jnp.int32),
            pltpu.VMEM((2, CHUNK, HIDDEN), f32),
            pltpu.VMEM((2, CHUNK, HIDDEN), f32),
            pltpu.VMEM((2, CHUNK // 2, 2 * HIDDEN), f32),
            pltpu.VMEM((2, CHUNK), f32),
            pltpu.VMEM((2, CHUNK), f32),
            pltpu.VMEM((N_CLASSES, HIDDEN), f32),
            pltpu.VMEM((L,), f32),
            pltpu.VMEM((N_NODES,), f32),
            pltpu.SemaphoreType.DMA,
            pltpu.SemaphoreType.DMA,
            pltpu.SemaphoreType.DMA,
            pltpu.SemaphoreType.DMA,
        ],
    )(_sc_edge_kernel)
    return kern(A, B, C, src, dst, w2t, b2)


def kernel(x, edge_index, edge_attr, W1, b1, W2, b2):
    src = edge_index[0].astype(jnp.int32)
    dst = edge_index[1].astype(jnp.int32)
    w1s = W1[:D_FEAT]
    w1d = W1[D_FEAT:2 * D_FEAT]
    w1e = W1[2 * D_FEAT:]

    A, B = _node_proj(x, w1s, w1d)
    ea2 = edge_attr.reshape(N_EDGES // 2, 2 * D_EDGE)
    w1e2 = jnp.zeros((2 * D_EDGE, 2 * HIDDEN), jnp.float32)
    w1e2 = w1e2.at[:D_EDGE, :HIDDEN].set(w1e)
    w1e2 = w1e2.at[D_EDGE:, HIDDEN:].set(w1e)
    b1row2 = jnp.concatenate([b1, b1]).reshape(1, 2 * HIDDEN)
    C = _edge_proj(ea2, w1e2, b1row2)
    b2pad = jnp.pad(b2.astype(jnp.float32), (0, L - N_CLASSES))
    p0, p1, partials = _sc_edge(A, B, C, src, dst,
                                W2.T.astype(jnp.float32), b2pad)
    node_best = _merge_partials(
        partials.reshape(NW, 8, N_NODES // 8)).reshape(N_NODES)
    edge_pred = jnp.stack([p0, p1], axis=1)
    return edge_pred, node_best


# speedup vs baseline: 1.5014x; 1.0950x over previous
import functools

import numpy as np

import jax
import jax.numpy as jnp
from jax import lax
from jax.experimental import pallas as pl
from jax.experimental.pallas import tpu as pltpu, tpu_sc as plsc

N_NODES = 10000
N_EDGES = 320000
D_FEAT = 128
D_EDGE = 16
HIDDEN = 64
N_CLASSES = 2

def _lane_take(v, idx):
    dn = lax.GatherDimensionNumbers(
        offset_dims=(), collapsed_slice_dims=(0,), start_index_map=(0,))
    return lax.gather(v, idx[:, None], dn, (1,),
                      mode=lax.GatherScatterMode.PROMISE_IN_BOUNDS)


def _allsum(v, iota):
    for sh in (8, 4, 2, 1):
        v = v + _lane_take(v, jnp.bitwise_xor(iota, sh))
    return v


NW = 32
EDGES_PER_TILE = N_EDGES // NW
CHUNK = 80
N_CHUNKS = EDGES_PER_TILE // CHUNK
L = 16
DUPW = 1024


def _node_proj_body(x_ref, w1s_ref, w1d_ref, a_ref, b_ref):
    xv = x_ref[...]
    a_ref[...] = jnp.dot(xv, w1s_ref[...], preferred_element_type=jnp.float32)
    b_ref[...] = jnp.dot(xv, w1d_ref[...], preferred_element_type=jnp.float32)


def _node_proj(x, w1s, w1d):
    blk = 2000
    grid = (N_NODES // blk,)
    return pl.pallas_call(
        _node_proj_body,
        grid=grid,
        in_specs=[
            pl.BlockSpec((blk, D_FEAT), lambda i: (i, 0)),
            pl.BlockSpec((D_FEAT, HIDDEN), lambda i: (0, 0)),
            pl.BlockSpec((D_FEAT, HIDDEN), lambda i: (0, 0)),
        ],
        out_specs=[
            pl.BlockSpec((blk, HIDDEN), lambda i: (i, 0)),
            pl.BlockSpec((blk, HIDDEN), lambda i: (i, 0)),
        ],
        out_shape=[
            jax.ShapeDtypeStruct((N_NODES, HIDDEN), jnp.float32),
            jax.ShapeDtypeStruct((N_NODES, HIDDEN), jnp.float32),
        ],
    )(x, w1s, w1d)


def _edge_proj_body(ea_ref, w1e_ref, b1_ref, c_ref):
    c_ref[...] = (
        jnp.dot(ea_ref[...], w1e_ref[...], preferred_element_type=jnp.float32)
        + b1_ref[...]
    )


def _edge_proj(ea2, w1e2, b1row2):
    blk = 8000
    rows = ea2.shape[0]
    grid = (rows // blk,)
    return pl.pallas_call(
        _edge_proj_body,
        grid=grid,
        in_specs=[
            pl.BlockSpec((blk, 2 * D_EDGE), lambda i: (i, 0)),
            pl.BlockSpec((2 * D_EDGE, 2 * HIDDEN), lambda i: (0, 0)),
            pl.BlockSpec((1, 2 * HIDDEN), lambda i: (0, 0)),
        ],
        out_specs=pl.BlockSpec((blk, 2 * HIDDEN), lambda i: (i, 0)),
        out_shape=jax.ShapeDtypeStruct((rows, 2 * HIDDEN), jnp.float32),
    )(ea2, w1e2, b1row2)


def _merge_body(pa_ref, pb_ref, o_ref):
    o_ref[...] = jnp.maximum(jnp.max(pa_ref[...], axis=0),
                             jnp.max(pb_ref[...], axis=0))


def _merge_partials(pa3, pb3):
    return pl.pallas_call(
        _merge_body,
        out_shape=jax.ShapeDtypeStruct((8, N_NODES // 8), jnp.float32),
    )(pa3, pb3)


def _make_sc_body(e_lo, ept, nch):
  def _sc_edge_kernel(a_hbm, b_hbm, c_hbm, src_hbm, dst_hbm, w2t_hbm, b2_hbm,
                      pred0_hbm, pred1_hbm, part_hbm,
                      srcall_v, dstall_v, a3, b3, c3, p03, p13,
                      w2_v, b2_v, best_v,
                      sem_in0, sem_in1, sem_out0, sem_out1):
    wid = lax.axis_index("s") * 2 + lax.axis_index("c")
    tile_base = wid * ept
    sem_in = (sem_in0, sem_in1)
    sem_out = (sem_out0, sem_out1)

    pltpu.sync_copy(w2t_hbm, w2_v)
    pltpu.sync_copy(b2_hbm, b2_v)
    pltpu.sync_copy(src_hbm.at[pl.ds(e_lo + tile_base, ept)], srcall_v)
    pltpu.sync_copy(dst_hbm.at[pl.ds(e_lo + tile_base, ept)], dstall_v)
    iota = lax.iota(jnp.int32, L)
    b2vec = b2_v[...]
    b2_0v = _lane_take(b2vec, jnp.zeros((L,), jnp.int32))
    b2_1v = _lane_take(b2vec, jnp.ones((L,), jnp.int32))
    w2c0 = [w2_v[0, pl.ds(q * L, L)] for q in range(HIDDEN // L)]
    w2c1 = [w2_v[1, pl.ds(q * L, L)] for q in range(HIDDEN // L)]

    ninf = jnp.full((L,), -jnp.inf, dtype=jnp.float32)

    def init_body(i, _):
        best_v[pl.ds(i * L, L)] = ninf
        return 0

    lax.fori_loop(0, N_NODES // L, init_body, 0)

    rot_idx = [jnp.bitwise_and(iota + r, L - 1) for r in range(1, L)]
    iotaL = iota * L

    def issue(g, bp):
        esl = pl.ds(g * CHUNK, CHUNK)
        pltpu.async_copy(a_hbm.at[srcall_v.at[esl]], a3.at[bp], sem_in[bp])
        pltpu.async_copy(b_hbm.at[dstall_v.at[esl]], b3.at[bp], sem_in[bp])
        pltpu.async_copy(
            c_hbm.at[pl.ds((tile_base + g * CHUNK) // 2, CHUNK // 2)],
            c3.at[bp], sem_in[bp])

    def drain_in(bp):
        dummy_f32 = a_hbm.at[pl.ds(0, CHUNK)]
        pltpu.make_async_copy(dummy_f32, a3.at[bp], sem_in[bp]).wait()
        pltpu.make_async_copy(dummy_f32, b3.at[bp], sem_in[bp]).wait()
        pltpu.make_async_copy(c_hbm.at[pl.ds(0, CHUNK // 2)], c3.at[bp],
                              sem_in[bp]).wait()

    def drain_out(bp):
        dummy = pred0_hbm.at[pl.ds(0, CHUNK)]
        pltpu.make_async_copy(dummy, p03.at[bp], sem_out[bp]).wait()
        pltpu.make_async_copy(dummy, p13.at[bp], sem_out[bp]).wait()

    def compute(g, bp):
        @pl.when(g >= 2)
        def _():
            drain_out(bp)

        for j in range(CHUNK // L):
            def edge_body(i, carry):
                p0vec, p1vec = carry
                e = j * L + i
                erow = lax.shift_right_logical(e, 1)
                ecol = jnp.bitwise_and(e, 1) * HIDDEN
                acc0 = jnp.zeros((L,), jnp.float32)
                acc1 = jnp.zeros((L,), jnp.float32)
                for q in range(HIDDEN // L):
                    sl = pl.ds(q * L, L)
                    h = jnp.maximum(
                        a3[bp, e, sl] + b3[bp, e, sl]
                        + c3[bp, erow, pl.ds(ecol + q * L, L)],
                        0.0)
                    acc0 = acc0 + h * w2c0[q]
                    acc1 = acc1 + h * w2c1[q]
                m = iota == i
                p0vec = jnp.where(m, _allsum(acc0, iota), p0vec)
                p1vec = jnp.where(m, _allsum(acc1, iota), p1vec)
                return p0vec, p1vec

            zero = jnp.zeros((L,), jnp.float32)
            p0g, p1g = lax.fori_loop(0, L, edge_body, (zero, zero))
            p0g = p0g + b2_0v
            p1g = p1g + b2_1v
            sl = pl.ds(j * L, L)
            p03[bp, sl] = p0g
            p13[bp, sl] = p1g

            score = 1.0 / (1.0 + jnp.exp(-p1g))
            k = dstall_v[pl.ds(g * CHUNK + j * L, L)]
            v = score
            dup_earlier = jnp.zeros((L,), jnp.bool_)
            for r in range(1, L):
                kr = _lane_take(k, rot_idx[r - 1])
                vr = _lane_take(v, rot_idx[r - 1])
                same = kr == k
                v = jnp.where(same, jnp.maximum(v, vr), v)
                dup_earlier = dup_earlier | (same & (iota + r >= L))
            old = plsc.load_gather(best_v, [k])
            plsc.store_scatter(best_v, [k], jnp.maximum(old, v),
                               mask=jnp.logical_not(dup_earlier))

        base = tile_base + g * CHUNK
        pltpu.async_copy(p03.at[bp], pred0_hbm.at[pl.ds(base, CHUNK)],
                         sem_out[bp])
        pltpu.async_copy(p13.at[bp], pred1_hbm.at[pl.ds(base, CHUNK)],
                         sem_out[bp])

    issue(0, 0)

    def pair_body(kk, _):
        g = 2 * kk
        issue(g + 1, 1)
        drain_in(0)
        compute(g, 0)
        issue(g + 2, 0)
        drain_in(1)
        compute(g + 1, 1)
        return 0

    lax.fori_loop(0, (nch - 1) // 2, pair_body, 0)
    if nch % 2 == 1:
        drain_in(0)
        compute(nch - 1, 0)
    else:
        issue(nch - 1, 1)
        drain_in(0)
        compute(nch - 2, 0)
        drain_in(1)
        compute(nch - 1, 1)
    drain_out(0)
    drain_out(1)
    pltpu.sync_copy(best_v, part_hbm.at[wid])

  return _sc_edge_kernel


def _sc_edge(A, B, C, src, dst, w2t, b2, e_lo, n_e):
    mesh = plsc.VectorSubcoreMesh(core_axis_name="c", subcore_axis_name="s")
    f32 = jnp.float32
    ept = n_e // NW
    nch = ept // CHUNK
    kern = functools.partial(
        pl.kernel,
        mesh=mesh,
        compiler_params=pltpu.CompilerParams(
            needs_layout_passes=False, use_tc_tiling_on_sc=False),
        out_type=[
            jax.ShapeDtypeStruct((n_e,), f32),
            jax.ShapeDtypeStruct((n_e,), f32),
            jax.ShapeDtypeStruct((NW, N_NODES), f32),
        ],
        scratch_types=[
            pltpu.VMEM((ept,), jnp.int32),
            pltpu.VMEM((ept,), jnp.int32),
            pltpu.VMEM((2, CHUNK, HIDDEN), f32),
            pltpu.VMEM((2, CHUNK, HIDDEN), f32),
            pltpu.VMEM((2, CHUNK // 2, 2 * HIDDEN), f32),
            pltpu.VMEM((2, CHUNK), f32),
            pltpu.VMEM((2, CHUNK), f32),
            pltpu.VMEM((N_CLASSES, HIDDEN), f32),
            pltpu.VMEM((L,), f32),
            pltpu.VMEM((N_NODES,), f32),
            pltpu.SemaphoreType.DMA,
            pltpu.SemaphoreType.DMA,
            pltpu.SemaphoreType.DMA,
            pltpu.SemaphoreType.DMA,
        ],
    )(_make_sc_body(e_lo, ept, nch))
    return kern(A, B, C, src, dst, w2t, b2)


def kernel(x, edge_index, edge_attr, W1, b1, W2, b2):
    src = edge_index[0].astype(jnp.int32)
    dst = edge_index[1].astype(jnp.int32)
    w1s = W1[:D_FEAT]
    w1d = W1[D_FEAT:2 * D_FEAT]
    w1e = W1[2 * D_FEAT:]

    A, B = _node_proj(x, w1s, w1d)
    w1e2 = jnp.zeros((2 * D_EDGE, 2 * HIDDEN), jnp.float32)
    w1e2 = w1e2.at[:D_EDGE, :HIDDEN].set(w1e)
    w1e2 = w1e2.at[D_EDGE:, HIDDEN:].set(w1e)
    b1row2 = jnp.concatenate([b1, b1]).reshape(1, 2 * HIDDEN)
    b2pad = jnp.pad(b2.astype(jnp.float32), (0, L - N_CLASSES))
    w2t = W2.T.astype(jnp.float32)

    E_A = 192000
    ea2_a = edge_attr[:E_A].reshape(E_A // 2, 2 * D_EDGE)
    ea2_b = edge_attr[E_A:].reshape((N_EDGES - E_A) // 2, 2 * D_EDGE)
    C_a = _edge_proj(ea2_a, w1e2, b1row2)
    C_b = _edge_proj(ea2_b, w1e2, b1row2)
    p0a, p1a, part_a = _sc_edge(A, B, C_a, src, dst, w2t, b2pad, 0, E_A)
    p0b, p1b, part_b = _sc_edge(A, B, C_b, src, dst, w2t, b2pad,
                                E_A, N_EDGES - E_A)
    node_best = _merge_partials(
        part_a.reshape(NW, 8, N_NODES // 8),
        part_b.reshape(NW, 8, N_NODES // 8)).reshape(N_NODES)
    edge_pred = jnp.stack([jnp.concatenate([p0a, p0b]),
                           jnp.concatenate([p1a, p1b])], axis=1)
    return edge_pred, node_best


# speedup vs baseline: 1.5037x; 1.0015x over previous
import functools

import numpy as np

import jax
import jax.numpy as jnp
from jax import lax
from jax.experimental import pallas as pl
from jax.experimental.pallas import tpu as pltpu, tpu_sc as plsc

N_NODES = 10000
N_EDGES = 320000
D_FEAT = 128
D_EDGE = 16
HIDDEN = 64
N_CLASSES = 2

def _lane_take(v, idx):
    dn = lax.GatherDimensionNumbers(
        offset_dims=(), collapsed_slice_dims=(0,), start_index_map=(0,))
    return lax.gather(v, idx[:, None], dn, (1,),
                      mode=lax.GatherScatterMode.PROMISE_IN_BOUNDS)


def _allsum(v, iota):
    for sh in (8, 4, 2, 1):
        v = v + _lane_take(v, jnp.bitwise_xor(iota, sh))
    return v


NW = 32
EDGES_PER_TILE = N_EDGES // NW
CHUNK = 80
N_CHUNKS = EDGES_PER_TILE // CHUNK
L = 16
DUPW = 1024


def _node_proj_body(x_ref, w1s_ref, w1d_ref, a_ref, b_ref):
    xv = x_ref[...]
    a_ref[...] = jnp.dot(xv, w1s_ref[...], preferred_element_type=jnp.float32)
    b_ref[...] = jnp.dot(xv, w1d_ref[...], preferred_element_type=jnp.float32)


def _node_proj(x, w1s, w1d):
    blk = 2000
    grid = (N_NODES // blk,)
    return pl.pallas_call(
        _node_proj_body,
        grid=grid,
        in_specs=[
            pl.BlockSpec((blk, D_FEAT), lambda i: (i, 0)),
            pl.BlockSpec((D_FEAT, HIDDEN), lambda i: (0, 0)),
            pl.BlockSpec((D_FEAT, HIDDEN), lambda i: (0, 0)),
        ],
        out_specs=[
            pl.BlockSpec((blk, HIDDEN), lambda i: (i, 0)),
            pl.BlockSpec((blk, HIDDEN), lambda i: (i, 0)),
        ],
        out_shape=[
            jax.ShapeDtypeStruct((N_NODES, HIDDEN), jnp.float32),
            jax.ShapeDtypeStruct((N_NODES, HIDDEN), jnp.float32),
        ],
    )(x, w1s, w1d)


def _edge_proj_body(ea_ref, w1e_ref, b1_ref, c_ref):
    c_ref[...] = (
        jnp.dot(ea_ref[...], w1e_ref[...], preferred_element_type=jnp.float32)
        + b1_ref[...]
    )


def _edge_proj(ea2, w1e2, b1row2):
    blk = 8000
    rows = ea2.shape[0]
    grid = (rows // blk,)
    return pl.pallas_call(
        _edge_proj_body,
        grid=grid,
        in_specs=[
            pl.BlockSpec((blk, 2 * D_EDGE), lambda i: (i, 0)),
            pl.BlockSpec((2 * D_EDGE, 2 * HIDDEN), lambda i: (0, 0)),
            pl.BlockSpec((1, 2 * HIDDEN), lambda i: (0, 0)),
        ],
        out_specs=pl.BlockSpec((blk, 2 * HIDDEN), lambda i: (i, 0)),
        out_shape=jax.ShapeDtypeStruct((rows, 2 * HIDDEN), jnp.float32),
    )(ea2, w1e2, b1row2)


def _merge_body(pa_ref, pb_ref, o_ref):
    o_ref[...] = jnp.maximum(jnp.max(pa_ref[...], axis=0),
                             jnp.max(pb_ref[...], axis=0))


def _merge_partials(pa3, pb3):
    return pl.pallas_call(
        _merge_body,
        out_shape=jax.ShapeDtypeStruct((8, N_NODES // 8), jnp.float32),
    )(pa3, pb3)


def _make_sc_body(e_lo, ept, nch):
  def _sc_edge_kernel(a_hbm, b_hbm, c_hbm, src_hbm, dst_hbm, w2t_hbm, b2_hbm,
                      pred0_hbm, pred1_hbm, part_hbm,
                      srcall_v, dstall_v, a3, b3, c3, p03, p13,
                      w2_v, b2_v, best_v,
                      sem_in0, sem_in1, sem_out0, sem_out1):
    wid = lax.axis_index("s") * 2 + lax.axis_index("c")
    tile_base = wid * ept
    sem_in = (sem_in0, sem_in1)
    sem_out = (sem_out0, sem_out1)

    pltpu.sync_copy(w2t_hbm, w2_v)
    pltpu.sync_copy(b2_hbm, b2_v)
    pltpu.sync_copy(src_hbm.at[pl.ds(e_lo + tile_base, ept)], srcall_v)
    pltpu.sync_copy(dst_hbm.at[pl.ds(e_lo + tile_base, ept)], dstall_v)
    iota = lax.iota(jnp.int32, L)
    b2vec = b2_v[...]
    b2_0v = _lane_take(b2vec, jnp.zeros((L,), jnp.int32))
    b2_1v = _lane_take(b2vec, jnp.ones((L,), jnp.int32))
    w2c0 = [w2_v[0, pl.ds(q * L, L)] for q in range(HIDDEN // L)]
    w2c1 = [w2_v[1, pl.ds(q * L, L)] for q in range(HIDDEN // L)]

    ninf = jnp.full((L,), -jnp.inf, dtype=jnp.float32)

    def init_body(i, _):
        best_v[pl.ds(i * L, L)] = ninf
        return 0

    lax.fori_loop(0, N_NODES // L, init_body, 0)

    rot_idx = [jnp.bitwise_and(iota + r, L - 1) for r in range(1, L)]
    iotaL = iota * L

    def issue(g, bp):
        esl = pl.ds(g * CHUNK, CHUNK)
        pltpu.async_copy(a_hbm.at[srcall_v.at[esl]], a3.at[bp], sem_in[bp])
        pltpu.async_copy(b_hbm.at[dstall_v.at[esl]], b3.at[bp], sem_in[bp])
        pltpu.async_copy(
            c_hbm.at[pl.ds((tile_base + g * CHUNK) // 2, CHUNK // 2)],
            c3.at[bp], sem_in[bp])

    def drain_in(bp):
        dummy_f32 = a_hbm.at[pl.ds(0, CHUNK)]
        pltpu.make_async_copy(dummy_f32, a3.at[bp], sem_in[bp]).wait()
        pltpu.make_async_copy(dummy_f32, b3.at[bp], sem_in[bp]).wait()
        pltpu.make_async_copy(c_hbm.at[pl.ds(0, CHUNK // 2)], c3.at[bp],
                              sem_in[bp]).wait()

    def drain_out(bp):
        dummy = pred0_hbm.at[pl.ds(0, CHUNK)]
        pltpu.make_async_copy(dummy, p03.at[bp], sem_out[bp]).wait()
        pltpu.make_async_copy(dummy, p13.at[bp], sem_out[bp]).wait()

    def compute(g, bp):
        @pl.when(g >= 2)
        def _():
            drain_out(bp)

        for j in range(CHUNK // L):
            def edge_body(i, carry):
                p0vec, p1vec = carry
                e = j * L + i
                erow = lax.shift_right_logical(e, 1)
                ecol = jnp.bitwise_and(e, 1) * HIDDEN
                acc0 = jnp.zeros((L,), jnp.float32)
                acc1 = jnp.zeros((L,), jnp.float32)
                for q in range(HIDDEN // L):
                    sl = pl.ds(q * L, L)
                    h = jnp.maximum(
                        a3[bp, e, sl] + b3[bp, e, sl]
                        + c3[bp, erow, pl.ds(ecol + q * L, L)],
                        0.0)
                    acc0 = acc0 + h * w2c0[q]
                    acc1 = acc1 + h * w2c1[q]
                m = iota == i
                p0vec = jnp.where(m, _allsum(acc0, iota), p0vec)
                p1vec = jnp.where(m, _allsum(acc1, iota), p1vec)
                return p0vec, p1vec

            zero = jnp.zeros((L,), jnp.float32)
            p0g, p1g = lax.fori_loop(0, L, edge_body, (zero, zero))
            p0g = p0g + b2_0v
            p1g = p1g + b2_1v
            sl = pl.ds(j * L, L)
            p03[bp, sl] = p0g
            p13[bp, sl] = p1g

            score = 1.0 / (1.0 + jnp.exp(-p1g))
            k = dstall_v[pl.ds(g * CHUNK + j * L, L)]
            v = score
            dup_earlier = jnp.zeros((L,), jnp.bool_)
            for r in range(1, L):
                kr = _lane_take(k, rot_idx[r - 1])
                vr = _lane_take(v, rot_idx[r - 1])
                same = kr == k
                v = jnp.where(same, jnp.maximum(v, vr), v)
                dup_earlier = dup_earlier | (same & (iota + r >= L))
            old = plsc.load_gather(best_v, [k])
            plsc.store_scatter(best_v, [k], jnp.maximum(old, v),
                               mask=jnp.logical_not(dup_earlier))

        base = tile_base + g * CHUNK
        pltpu.async_copy(p03.at[bp], pred0_hbm.at[pl.ds(base, CHUNK)],
                         sem_out[bp])
        pltpu.async_copy(p13.at[bp], pred1_hbm.at[pl.ds(base, CHUNK)],
                         sem_out[bp])

    issue(0, 0)

    def pair_body(kk, _):
        g = 2 * kk
        issue(g + 1, 1)
        drain_in(0)
        compute(g, 0)
        issue(g + 2, 0)
        drain_in(1)
        compute(g + 1, 1)
        return 0

    lax.fori_loop(0, (nch - 1) // 2, pair_body, 0)
    if nch % 2 == 1:
        drain_in(0)
        compute(nch - 1, 0)
    else:
        issue(nch - 1, 1)
        drain_in(0)
        compute(nch - 2, 0)
        drain_in(1)
        compute(nch - 1, 1)
    drain_out(0)
    drain_out(1)
    pltpu.sync_copy(best_v, part_hbm.at[wid])

  return _sc_edge_kernel


def _sc_edge(A, B, C, src, dst, w2t, b2, e_lo, n_e):
    mesh = plsc.VectorSubcoreMesh(core_axis_name="c", subcore_axis_name="s")
    f32 = jnp.float32
    ept = n_e // NW
    nch = ept // CHUNK
    kern = functools.partial(
        pl.kernel,
        mesh=mesh,
        compiler_params=pltpu.CompilerParams(
            needs_layout_passes=False, use_tc_tiling_on_sc=False),
        out_type=[
            jax.ShapeDtypeStruct((n_e,), f32),
            jax.ShapeDtypeStruct((n_e,), f32),
            jax.ShapeDtypeStruct((NW, N_NODES), f32),
        ],
        scratch_types=[
            pltpu.VMEM((ept,), jnp.int32),
            pltpu.VMEM((ept,), jnp.int32),
            pltpu.VMEM((2, CHUNK, HIDDEN), f32),
            pltpu.VMEM((2, CHUNK, HIDDEN), f32),
            pltpu.VMEM((2, CHUNK // 2, 2 * HIDDEN), f32),
            pltpu.VMEM((2, CHUNK), f32),
            pltpu.VMEM((2, CHUNK), f32),
            pltpu.VMEM((N_CLASSES, HIDDEN), f32),
            pltpu.VMEM((L,), f32),
            pltpu.VMEM((N_NODES,), f32),
            pltpu.SemaphoreType.DMA,
            pltpu.SemaphoreType.DMA,
            pltpu.SemaphoreType.DMA,
            pltpu.SemaphoreType.DMA,
        ],
    )(_make_sc_body(e_lo, ept, nch))
    return kern(A, B, C, src, dst, w2t, b2)


def kernel(x, edge_index, edge_attr, W1, b1, W2, b2):
    src = edge_index[0].astype(jnp.int32)
    dst = edge_index[1].astype(jnp.int32)
    w1s = W1[:D_FEAT]
    w1d = W1[D_FEAT:2 * D_FEAT]
    w1e = W1[2 * D_FEAT:]

    A, B = _node_proj(x, w1s, w1d)
    w1e2 = jnp.zeros((2 * D_EDGE, 2 * HIDDEN), jnp.float32)
    w1e2 = w1e2.at[:D_EDGE, :HIDDEN].set(w1e)
    w1e2 = w1e2.at[D_EDGE:, HIDDEN:].set(w1e)
    b1row2 = jnp.concatenate([b1, b1]).reshape(1, 2 * HIDDEN)
    b2pad = jnp.pad(b2.astype(jnp.float32), (0, L - N_CLASSES))
    w2t = W2.T.astype(jnp.float32)

    E_A = 128000
    ea2_a = edge_attr[:E_A].reshape(E_A // 2, 2 * D_EDGE)
    ea2_b = edge_attr[E_A:].reshape((N_EDGES - E_A) // 2, 2 * D_EDGE)
    C_a = _edge_proj(ea2_a, w1e2, b1row2)
    C_b = _edge_proj(ea2_b, w1e2, b1row2)
    p0a, p1a, part_a = _sc_edge(A, B, C_a, src, dst, w2t, b2pad, 0, E_A)
    p0b, p1b, part_b = _sc_edge(A, B, C_b, src, dst, w2t, b2pad,
                                E_A, N_EDGES - E_A)
    node_best = _merge_partials(
        part_a.reshape(NW, 8, N_NODES // 8),
        part_b.reshape(NW, 8, N_NODES // 8)).reshape(N_NODES)
    edge_pred = jnp.stack([jnp.concatenate([p0a, p0b]),
                           jnp.concatenate([p1a, p1b])], axis=1)
    return edge_pred, node_best
